# baseline (speedup 1.0000x reference)
import functools

import numpy as np
import jax
import jax.numpy as jnp
from jax import lax
from jax.experimental import pallas as pl
from jax.experimental.pallas import tpu as pltpu
from jax.experimental.pallas import tpu_sc as plsc

_B = 4096
_NT = 26
_V = 100000
_D = 64
_R = _NT * _B
_NW = 32
_RW = _R // _NW
_CHUNK = 128
_NCH = _RW // _CHUNK

_BB = 256
_NI = _NT + 1

_ZOFF = np.zeros(_NI, dtype=np.int64)
for _i in range(1, _NT):
    _ZOFF[_i + 1] = _ZOFF[_i] + (-(-_i // 8) * 8)
_ZROWS = int(_ZOFF[_NT] + (-(-_NT // 8) * 8))
_RROWS = _D + _ZROWS

_RCOLS = np.full(_RROWS, -1, dtype=np.int64)
_RCOLS[:_D] = np.arange(_D)
for _i in range(1, _NI):
    _s = _D + _i * (_i - 1) // 2
    _o = _D + int(_ZOFF[_i])
    _RCOLS[_o:_o + _i] = np.arange(_s, _s + _i)
_RVALID = (_RCOLS >= 0)
_RCOLS_SAFE = np.where(_RVALID, _RCOLS, 0)


def _sc_gather(emb, tidx3, ridx3):
    mesh = plsc.VectorSubcoreMesh(core_axis_name="c", subcore_axis_name="s")
    info = plsc.get_sparse_core_info()
    nc = info.num_cores

    @functools.partial(
        pl.kernel,
        mesh=mesh,
        out_type=jax.ShapeDtypeStruct((_R, _D), jnp.float32),
        scratch_types=[
            pltpu.VMEM((_NCH, _CHUNK), jnp.int32),
            pltpu.VMEM((_NCH, _CHUNK), jnp.int32),
            pltpu.VMEM((2, _CHUNK, _D), jnp.float32),
            pltpu.SemaphoreType.DMA((2,)),
        ],
    )
    def k(emb_hbm, tidx_hbm, ridx_hbm, out_hbm, tidx_v, ridx_v, rows_v, sems):
        wid = lax.axis_index("s") * nc + lax.axis_index("c")
        base = wid * _RW
        pltpu.sync_copy(tidx_hbm.at[wid], tidx_v)
        pltpu.sync_copy(ridx_hbm.at[wid], ridx_v)

        def enqueue(j, b):
            def grp(g, carry):
                tv = tidx_v[j, pl.ds(g * 16, 16)]
                rv = ridx_v[j, pl.ds(g * 16, 16)]
                for i in range(16):
                    pltpu.async_copy(
                        emb_hbm.at[tv[i]].at[pl.ds(rv[i], 1)],
                        rows_v.at[b].at[pl.ds(g * 16 + i, 1)], sems.at[b])
                return carry
            lax.fori_loop(0, _CHUNK // 16, grp, 0)

        def drain(j, b):
            def row(r, carry):
                pltpu.make_async_copy(
                    emb_hbm.at[0].at[pl.ds(0, 1)], rows_v.at[b].at[pl.ds(0, 1)],
                    sems.at[b]).wait()
                return carry
            lax.fori_loop(0, _CHUNK, row, 0)
            pltpu.sync_copy(rows_v.at[b],
                            out_hbm.at[pl.ds(base + j * _CHUNK, _CHUNK)])

        enqueue(0, 0)

        def body(j, carry):
            b = j % 2
            enqueue(j + 1, 1 - b)
            drain(j, b)
            return carry

        lax.fori_loop(0, _NCH - 1, body, 0, unroll=2)
        drain(_NCH - 1, (_NCH - 1) % 2)

    return k(emb, tidx3, ridx3)


def _tc_body(dxT, ly, wb0, b0, wb1, b1, wb2, b2,
             w0p, bt0, wt1, bt1, wt2, bt2, out, t_scr, rt_scr):
    f32 = jnp.float32
    h = jnp.maximum(jnp.dot(wb0[...], dxT[...], preferred_element_type=f32) + b0[...], 0.0)
    h = jnp.maximum(jnp.dot(wb1[...], h, preferred_element_type=f32) + b1[...], 0.0)
    xT = jnp.maximum(jnp.dot(wb2[...], h, preferred_element_type=f32) + b2[...], 0.0)

    t_scr[0] = xT
    for t in range(_NT):
        t_scr[1 + t] = ly[t].T

    rt_scr[...] = jnp.zeros((_RROWS, _BB), f32)
    rt_scr[0:_D] = xT
    for i in range(1, _NI):
        zi = jnp.sum(t_scr[i] * t_scr[0:i], axis=1)
        o = _D + int(_ZOFF[i])
        rt_scr[o:o + i, :] = zi

    h = jnp.maximum(jnp.dot(w0p[...], rt_scr[...], preferred_element_type=f32) + bt0[...], 0.0)
    h = jnp.maximum(jnp.dot(wt1[...], h, preferred_element_type=f32) + bt1[...], 0.0)
    y = jnp.dot(wt2[...], h, preferred_element_type=f32) + bt2[...]
    out[...] = jax.nn.sigmoid(y)


def kernel(dense_x, lS_o, lS_i, emb, W_bot0, b_bot0, W_bot1, b_bot1, W_bot2, b_bot2,
           W_top0, b_top0, W_top1, b_top1, W_top2, b_top2):
    del lS_o
    f32 = jnp.float32

    ridx = lS_i.astype(jnp.int32)
    tidx = jnp.broadcast_to(jnp.arange(_NT, dtype=jnp.int32)[:, None], (_NT, _B))
    ridx3 = ridx.reshape(_NW, _NCH, _CHUNK)
    tidx3 = tidx.reshape(_NW, _NCH, _CHUNK)
    ly = _sc_gather(emb, tidx3, ridx3).reshape(_NT, _B, _D)

    dxT = dense_x.T
    w0p = W_top0[:, _RCOLS_SAFE] * jnp.asarray(_RVALID, f32)[None, :]
    b0, b1, b2 = b_bot0[:, None], b_bot1[:, None], b_bot2[:, None]
    bt0, bt1, bt2 = b_top0[:, None], b_top1[:, None], b_top2[:, None]

    grid = (_B // _BB,)
    const = lambda shape: pl.BlockSpec(shape, lambda i: tuple(0 for _ in shape))
    out2 = pl.pallas_call(
        _tc_body,
        grid=grid,
        in_specs=[
            pl.BlockSpec((13, _BB), lambda i: (0, i)),
            pl.BlockSpec((_NT, _BB, _D), lambda i: (0, i, 0)),
            const((512, 13)), const((512, 1)),
            const((256, 512)), const((256, 1)),
            const((_D, 256)), const((_D, 1)),
            const((512, _RROWS)), const((512, 1)),
            const((256, 512)), const((256, 1)),
            const((1, 256)), const((1, 1)),
        ],
        out_specs=pl.BlockSpec((1, _BB), lambda i: (0, i)),
        out_shape=jax.ShapeDtypeStruct((1, _B), f32),
        scratch_shapes=[
            pltpu.VMEM((_NI, _D, _BB), f32),
            pltpu.VMEM((_RROWS, _BB), f32),
        ],
    )(dxT, ly,
      W_bot0, b0, W_bot1, b1, W_bot2, b2,
      w0p, bt0, W_top1, bt1, W_top2, bt2)
    return out2.reshape(_B, 1)


# speedup vs baseline: 1.6720x; 1.6720x over previous
import functools

import numpy as np
import jax
import jax.numpy as jnp
from jax import lax
from jax.experimental import pallas as pl
from jax.experimental.pallas import tpu as pltpu
from jax.experimental.pallas import tpu_sc as plsc

_B = 4096
_NT = 26
_V = 100000
_D = 64
_R = _NT * _B
_NW = 32
_RW = _R // _NW
_CHUNK = 128
_NCH = _RW // _CHUNK

_BB = 256
_NI = _NT + 1

_ZOFF = np.zeros(_NI, dtype=np.int64)
for _i in range(1, _NT):
    _ZOFF[_i + 1] = _ZOFF[_i] + (-(-_i // 8) * 8)
_ZROWS = int(_ZOFF[_NT] + (-(-_NT // 8) * 8))
_RROWS = _D + _ZROWS

_RCOLS = np.full(_RROWS, -1, dtype=np.int64)
_RCOLS[:_D] = np.arange(_D)
for _i in range(1, _NI):
    _s = _D + _i * (_i - 1) // 2
    _o = _D + int(_ZOFF[_i])
    _RCOLS[_o:_o + _i] = np.arange(_s, _s + _i)
_RVALID = (_RCOLS >= 0)
_RCOLS_SAFE = np.where(_RVALID, _RCOLS, 0)


def _sc_gather(emb_flat, idx3):
    mesh = plsc.VectorSubcoreMesh(core_axis_name="c", subcore_axis_name="s")
    info = plsc.get_sparse_core_info()
    nc = info.num_cores

    @functools.partial(
        pl.kernel,
        mesh=mesh,
        out_type=jax.ShapeDtypeStruct((_R, _D), jnp.float32),
        scratch_types=[
            pltpu.VMEM((_NCH, _CHUNK), jnp.int32),
            pltpu.VMEM((2, _CHUNK, _D), jnp.float32),
            pltpu.SemaphoreType.DMA((2,)),
        ],
    )
    def k(emb_hbm, idx_hbm, out_hbm, idx_v, rows_v, sems):
        wid = lax.axis_index("s") * nc + lax.axis_index("c")
        base = wid * _RW
        pltpu.sync_copy(idx_hbm.at[wid], idx_v)

        def enqueue(j, b):
            def grp(g, carry):
                v = idx_v[j, pl.ds(g * 16, 16)]
                for i in range(16):
                    pltpu.async_copy(
                        emb_hbm.at[pl.ds(v[i], 1)],
                        rows_v.at[b].at[pl.ds(g * 16 + i, 1)], sems.at[b])
                return carry
            lax.fori_loop(0, _CHUNK // 16, grp, 0)

        def drain(j, b):
            def row(r, carry):
                pltpu.make_async_copy(
                    emb_hbm.at[pl.ds(0, 1)], rows_v.at[b].at[pl.ds(0, 1)],
                    sems.at[b]).wait()
                return carry
            lax.fori_loop(0, _CHUNK, row, 0)
            pltpu.sync_copy(rows_v.at[b],
                            out_hbm.at[pl.ds(base + j * _CHUNK, _CHUNK)])

        enqueue(0, 0)

        def body(j, carry):
            b = j % 2
            enqueue(j + 1, 1 - b)
            drain(j, b)
            return carry

        lax.fori_loop(0, _NCH - 1, body, 0, unroll=2)
        drain(_NCH - 1, (_NCH - 1) % 2)

    return k(emb_flat, idx3)


def _tc_body(dxT, ly, wb0, b0, wb1, b1, wb2, b2,
             w0p, bt0, wt1, bt1, wt2, bt2, out, t_scr, rt_scr):
    f32 = jnp.float32
    h = jnp.maximum(jnp.dot(wb0[...], dxT[...], preferred_element_type=f32) + b0[...], 0.0)
    h = jnp.maximum(jnp.dot(wb1[...], h, preferred_element_type=f32) + b1[...], 0.0)
    xT = jnp.maximum(jnp.dot(wb2[...], h, preferred_element_type=f32) + b2[...], 0.0)

    t_scr[0] = xT
    for t in range(_NT):
        t_scr[1 + t] = ly[t].T

    rt_scr[...] = jnp.zeros((_RROWS, _BB), f32)
    rt_scr[0:_D] = xT
    for i in range(1, _NI):
        zi = jnp.sum(t_scr[i] * t_scr[0:i], axis=1)
        o = _D + int(_ZOFF[i])
        rt_scr[o:o + i, :] = zi

    h = jnp.maximum(jnp.dot(w0p[...], rt_scr[...], preferred_element_type=f32) + bt0[...], 0.0)
    h = jnp.maximum(jnp.dot(wt1[...], h, preferred_element_type=f32) + bt1[...], 0.0)
    y = jnp.dot(wt2[...], h, preferred_element_type=f32) + bt2[...]
    out[...] = jax.nn.sigmoid(y)


def kernel(dense_x, lS_o, lS_i, emb, W_bot0, b_bot0, W_bot1, b_bot1, W_bot2, b_bot2,
           W_top0, b_top0, W_top1, b_top1, W_top2, b_top2):
    del lS_o
    f32 = jnp.float32

    emb_flat = emb.reshape(_NT * _V, _D)
    fi = lS_i.astype(jnp.int32) + (jnp.arange(_NT, dtype=jnp.int32) * _V)[:, None]
    idx3 = fi.reshape(_NW, _NCH, _CHUNK)
    ly = _sc_gather(emb_flat, idx3).reshape(_NT, _B, _D)

    dxT = dense_x.T
    w0p = W_top0[:, _RCOLS_SAFE] * jnp.asarray(_RVALID, f32)[None, :]
    b0, b1, b2 = b_bot0[:, None], b_bot1[:, None], b_bot2[:, None]
    bt0, bt1, bt2 = b_top0[:, None], b_top1[:, None], b_top2[:, None]

    grid = (_B // _BB,)
    const = lambda shape: pl.BlockSpec(shape, lambda i: tuple(0 for _ in shape))
    out2 = pl.pallas_call(
        _tc_body,
        grid=grid,
        in_specs=[
            pl.BlockSpec((13, _BB), lambda i: (0, i)),
            pl.BlockSpec((_NT, _BB, _D), lambda i: (0, i, 0)),
            const((512, 13)), const((512, 1)),
            const((256, 512)), const((256, 1)),
            const((_D, 256)), const((_D, 1)),
            const((512, _RROWS)), const((512, 1)),
            const((256, 512)), const((256, 1)),
            const((1, 256)), const((1, 1)),
        ],
        out_specs=pl.BlockSpec((1, _BB), lambda i: (0, i)),
        out_shape=jax.ShapeDtypeStruct((1, _B), f32),
        scratch_shapes=[
            pltpu.VMEM((_NI, _D, _BB), f32),
            pltpu.VMEM((_RROWS, _BB), f32),
        ],
    )(dxT, ly,
      W_bot0, b0, W_bot1, b1, W_bot2, b2,
      w0p, bt0, W_top1, bt1, W_top2, bt2)
    return out2.reshape(_B, 1)


# speedup vs baseline: 1.6871x; 1.0090x over previous
import functools

import numpy as np
import jax
import jax.numpy as jnp
from jax import lax
from jax.experimental import pallas as pl
from jax.experimental.pallas import tpu as pltpu
from jax.experimental.pallas import tpu_sc as plsc

_B = 4096
_NT = 26
_V = 100000
_D = 64
_R = _NT * _B
_NW = 32
_RW = _R // _NW
_CHUNK = 128
_NCH = _RW // _CHUNK

_BB = 512
_NI = _NT + 1

_ZOFF = np.zeros(_NI, dtype=np.int64)
for _i in range(1, _NT):
    _ZOFF[_i + 1] = _ZOFF[_i] + (-(-_i // 8) * 8)
_ZROWS = int(_ZOFF[_NT] + (-(-_NT // 8) * 8))
_RROWS = _D + _ZROWS

_RCOLS = np.full(_RROWS, -1, dtype=np.int64)
_RCOLS[:_D] = np.arange(_D)
for _i in range(1, _NI):
    _s = _D + _i * (_i - 1) // 2
    _o = _D + int(_ZOFF[_i])
    _RCOLS[_o:_o + _i] = np.arange(_s, _s + _i)
_RVALID = (_RCOLS >= 0)
_RCOLS_SAFE = np.where(_RVALID, _RCOLS, 0)


def _sc_gather(emb_flat, idx3):
    mesh = plsc.VectorSubcoreMesh(core_axis_name="c", subcore_axis_name="s")
    info = plsc.get_sparse_core_info()
    nc = info.num_cores

    @functools.partial(
        pl.kernel,
        mesh=mesh,
        out_type=jax.ShapeDtypeStruct((_R, _D), jnp.float32),
        scratch_types=[
            pltpu.VMEM((_NCH, _CHUNK), jnp.int32),
            pltpu.VMEM((2, _CHUNK, _D), jnp.float32),
            pltpu.SemaphoreType.DMA((2,)),
        ],
    )
    def k(emb_hbm, idx_hbm, out_hbm, idx_v, rows_v, sems):
        wid = lax.axis_index("s") * nc + lax.axis_index("c")
        base = wid * _RW
        pltpu.sync_copy(idx_hbm.at[wid], idx_v)

        def enqueue(j, b):
            def grp(g, carry):
                v = idx_v[j, pl.ds(g * 16, 16)]
                for i in range(16):
                    pltpu.async_copy(
                        emb_hbm.at[pl.ds(v[i], 1)],
                        rows_v.at[b].at[pl.ds(g * 16 + i, 1)], sems.at[b])
                return carry
            lax.fori_loop(0, _CHUNK // 16, grp, 0)

        def drain(j, b):
            def row(r, carry):
                pltpu.make_async_copy(
                    emb_hbm.at[pl.ds(0, 1)], rows_v.at[b].at[pl.ds(0, 1)],
                    sems.at[b]).wait()
                return carry
            lax.fori_loop(0, _CHUNK, row, 0)
            pltpu.sync_copy(rows_v.at[b],
                            out_hbm.at[pl.ds(base + j * _CHUNK, _CHUNK)])

        enqueue(0, 0)

        def body(j, carry):
            b = j % 2
            enqueue(j + 1, 1 - b)
            drain(j, b)
            return carry

        lax.fori_loop(0, _NCH - 1, body, 0, unroll=2)
        drain(_NCH - 1, (_NCH - 1) % 2)

    return k(emb_flat, idx3)


def _tc_body(dxT, ly, wb0, b0, wb1, b1, wb2, b2,
             w0p, bt0, wt1, bt1, wt2, bt2, out, t_scr, rt_scr):
    f32 = jnp.float32
    h = jnp.maximum(jnp.dot(wb0[...], dxT[...], preferred_element_type=f32) + b0[...], 0.0)
    h = jnp.maximum(jnp.dot(wb1[...], h, preferred_element_type=f32) + b1[...], 0.0)
    xT = jnp.maximum(jnp.dot(wb2[...], h, preferred_element_type=f32) + b2[...], 0.0)

    t_scr[0] = xT
    for t in range(_NT):
        t_scr[1 + t] = ly[t].T

    rt_scr[...] = jnp.zeros((_RROWS, _BB), f32)
    rt_scr[0:_D] = xT
    for i in range(1, _NI):
        zi = jnp.sum(t_scr[i] * t_scr[0:i], axis=1)
        o = _D + int(_ZOFF[i])
        rt_scr[o:o + i, :] = zi

    h = jnp.maximum(jnp.dot(w0p[...], rt_scr[...], preferred_element_type=f32) + bt0[...], 0.0)
    h = jnp.maximum(jnp.dot(wt1[...], h, preferred_element_type=f32) + bt1[...], 0.0)
    y = jnp.dot(wt2[...], h, preferred_element_type=f32) + bt2[...]
    out[...] = jax.nn.sigmoid(y)


def kernel(dense_x, lS_o, lS_i, emb, W_bot0, b_bot0, W_bot1, b_bot1, W_bot2, b_bot2,
           W_top0, b_top0, W_top1, b_top1, W_top2, b_top2):
    del lS_o
    f32 = jnp.float32

    emb_flat = emb.reshape(_NT * _V, _D)
    fi = lS_i.astype(jnp.int32) + (jnp.arange(_NT, dtype=jnp.int32) * _V)[:, None]
    idx3 = fi.reshape(_NW, _NCH, _CHUNK)
    ly = _sc_gather(emb_flat, idx3).reshape(_NT, _B, _D)

    dxT = dense_x.T
    w0p = W_top0[:, _RCOLS_SAFE] * jnp.asarray(_RVALID, f32)[None, :]
    b0, b1, b2 = b_bot0[:, None], b_bot1[:, None], b_bot2[:, None]
    bt0, bt1, bt2 = b_top0[:, None], b_top1[:, None], b_top2[:, None]

    grid = (_B // _BB,)
    const = lambda shape: pl.BlockSpec(shape, lambda i: tuple(0 for _ in shape))
    out2 = pl.pallas_call(
        _tc_body,
        grid=grid,
        in_specs=[
            pl.BlockSpec((13, _BB), lambda i: (0, i)),
            pl.BlockSpec((_NT, _BB, _D), lambda i: (0, i, 0)),
            const((512, 13)), const((512, 1)),
            const((256, 512)), const((256, 1)),
            const((_D, 256)), const((_D, 1)),
            const((512, _RROWS)), const((512, 1)),
            const((256, 512)), const((256, 1)),
            const((1, 256)), const((1, 1)),
        ],
        out_specs=pl.BlockSpec((1, _BB), lambda i: (0, i)),
        out_shape=jax.ShapeDtypeStruct((1, _B), f32),
        scratch_shapes=[
            pltpu.VMEM((_NI, _D, _BB), f32),
            pltpu.VMEM((_RROWS, _BB), f32),
        ],
    )(dxT, ly,
      W_bot0, b0, W_bot1, b1, W_bot2, b2,
      w0p, bt0, W_top1, bt1, W_top2, bt2)
    return out2.reshape(_B, 1)
